# Initial kernel scaffold; baseline (speedup 1.0000x reference)
import functools

import jax
import jax.numpy as jnp
from jax import lax
from jax.experimental import pallas as pl
from jax.experimental.pallas import tpu as pltpu
from jax.experimental.pallas import tpu_sc as plsc

NC = 2
NS = 16
NW = NC * NS
LANES = 16
CHUNK = 128


def _make_sc_agg(n_pad, d, steps, with_cnt):
  rows_per_tile = n_pad // NS
  mesh = plsc.VectorSubcoreMesh(
      core_axis_name="c", subcore_axis_name="s",
      num_cores=NC, num_subcores=NS)

  out_type = [jax.ShapeDtypeStruct((NC, n_pad, d), jnp.float32)]
  scratch = [
      pltpu.VMEM_SHARED((n_pad, d), jnp.float32),
      pltpu.VMEM((steps, CHUNK), jnp.int32),
      pltpu.VMEM((steps, CHUNK), jnp.int32),
      pltpu.VMEM((CHUNK, d), jnp.float32),
      pltpu.SemaphoreType.DMA,
  ]
  if with_cnt:
    out_type.append(jax.ShapeDtypeStruct((NW, n_pad), jnp.float32))
    scratch.append(pltpu.VMEM((n_pad,), jnp.float32))

  def body(feat, src3, dst3, z2d, *rest):
    if with_cnt:
      agg_out, cnt_out, agg_sh, src_v, dst_v, rows_v, sem, cnt_v = rest
    else:
      agg_out, agg_sh, src_v, dst_v, rows_v, sem = rest
    c = lax.axis_index("c")
    s = lax.axis_index("s")
    wid = s * NC + c

    pltpu.sync_copy(src3.at[wid], src_v)
    pltpu.sync_copy(dst3.at[wid], dst_v)
    pltpu.sync_copy(z2d, agg_sh.at[pl.ds(s * rows_per_tile, rows_per_tile)])
    plsc.subcore_barrier()

    if with_cnt:
      zeros16 = jnp.zeros((LANES,), jnp.float32)
      ones16 = jnp.ones((LANES,), jnp.float32)

      def zero_cnt(i, carry):
        cnt_v[pl.ds(i * LANES, LANES)] = zeros16
        return carry
      lax.fori_loop(0, n_pad // LANES, zero_cnt, 0)

      def cnt_step(j, carry):
        def cnt_lane(l, carry2):
          idx16 = dst_v[j, pl.ds(l * LANES, LANES)]
          plsc.addupdate_scatter(cnt_v, [idx16], ones16)
          return carry2
        return lax.fori_loop(0, CHUNK // LANES, cnt_lane, carry)
      lax.fori_loop(0, steps, cnt_step, 0)
      pltpu.sync_copy(cnt_v, cnt_out.at[wid])

    def step(j, carry):
      pltpu.async_copy(feat.at[src_v.at[j]], rows_v, sem).wait()
      pltpu.sync_copy(rows_v, agg_sh.at[dst_v.at[j]], add=True)
      return carry
    lax.fori_loop(0, steps, step, 0)

    plsc.subcore_barrier()
    sl = pl.ds(s * rows_per_tile, rows_per_tile)
    pltpu.sync_copy(agg_sh.at[sl], agg_out.at[c, sl])

  return pl.kernel(body, out_type=tuple(out_type), mesh=mesh,
                   scratch_types=tuple(scratch))


def _tc_body(relu, p0, p1, cp, x, wl, wr, b, out):
  cnt = jnp.sum(cp[...], axis=0)
  inv = 1.0 / jnp.maximum(cnt, 1.0)
  mean = (p0[...] + p1[...]) * inv[:, None]
  acc = jnp.dot(mean, wl[...], preferred_element_type=jnp.float32)
  acc = acc + jnp.dot(x[...], wr[...], preferred_element_type=jnp.float32)
  acc = acc + b[...]
  if relu:
    acc = jnp.maximum(acc, 0.0)
  out[...] = acc


def _make_tc_layer(n, d, relu, blk=1000):
  grid = n // blk
  return pl.pallas_call(
      functools.partial(_tc_body, relu),
      grid=(grid,),
      in_specs=[
          pl.BlockSpec((blk, d), lambda i: (i, 0)),
          pl.BlockSpec((blk, d), lambda i: (i, 0)),
          pl.BlockSpec((NW, blk), lambda i: (0, i)),
          pl.BlockSpec((blk, d), lambda i: (i, 0)),
          pl.BlockSpec((d, d), lambda i: (0, 0)),
          pl.BlockSpec((d, d), lambda i: (0, 0)),
          pl.BlockSpec((1, d), lambda i: (0, 0)),
      ],
      out_specs=pl.BlockSpec((blk, d), lambda i: (i, 0)),
      out_shape=jax.ShapeDtypeStruct((n, d), jnp.float32),
  )


def kernel(x, edge_index, W1l, W1r, b1, W2l, W2r, b2):
  n, d = x.shape
  e = edge_index.shape[1]
  n_pad = ((n + 1 + 2047) // 2048) * 2048
  rows_per_tile = n_pad // NS
  steps = -(-e // (NW * CHUNK))
  e_pad = steps * NW * CHUNK

  src = jnp.pad(edge_index[0], (0, e_pad - e)).reshape(NW, steps, CHUNK)
  dst = jnp.pad(edge_index[1], (0, e_pad - e),
                constant_values=n).reshape(NW, steps, CHUNK)
  z2d = jnp.zeros((rows_per_tile, d), jnp.float32)
  b1r = b1.reshape(1, d)
  b2r = b2.reshape(1, d)

  sc1 = _make_sc_agg(n_pad, d, steps, with_cnt=True)
  sc2 = _make_sc_agg(n_pad, d, steps, with_cnt=False)
  tc1 = _make_tc_layer(n, d, relu=True)
  tc2 = _make_tc_layer(n, d, relu=False)

  agg1, cnt = sc1(x, src, dst, z2d)
  h = tc1(agg1[0, :n], agg1[1, :n], cnt[:, :n], x, W1l, W1r, b1r)
  agg2 = sc2(h, src, dst, z2d)
  out = tc2(agg2[0, :n], agg2[1, :n], cnt[:, :n], h, W2l, W2r, b2r)
  return out


# speedup vs baseline: 3.3159x; 3.3159x over previous
import functools

import jax
import jax.numpy as jnp
from jax import lax
from jax.experimental import pallas as pl
from jax.experimental.pallas import tpu as pltpu
from jax.experimental.pallas import tpu_sc as plsc

NC = 2
NS = 16
NW = NC * NS
LANES = 16
CHUNK = 128
SUP = 16


def _make_sc_agg(n_pad, d, steps, with_cnt):
  rows_per_tile = n_pad // NS
  nsup = steps // SUP
  mesh = plsc.VectorSubcoreMesh(
      core_axis_name="c", subcore_axis_name="s",
      num_cores=NC, num_subcores=NS)

  out_type = [jax.ShapeDtypeStruct((NC, n_pad, d), jnp.float32)]
  scratch = [
      pltpu.VMEM_SHARED((n_pad, d), jnp.float32),
      pltpu.VMEM((SUP, CHUNK), jnp.int32),
      pltpu.VMEM((SUP, CHUNK), jnp.int32),
      pltpu.VMEM((CHUNK, d), jnp.float32),
      pltpu.SemaphoreType.DMA,
  ]
  if with_cnt:
    out_type.append(jax.ShapeDtypeStruct((NC * n_pad,), jnp.float32))
    scratch.append(pltpu.VMEM_SHARED((n_pad,), jnp.float32))
    scratch.append(pltpu.VMEM((CHUNK,), jnp.float32))

  def body(feat, src3, dst3, z2d, z1d, *rest):
    if with_cnt:
      agg_out, cnt_out, agg_sh, src_v, dst_v, rows_v, sem, cnt_sh, ones_v = rest
    else:
      agg_out, agg_sh, src_v, dst_v, rows_v, sem = rest
    c = lax.axis_index("c")
    s = lax.axis_index("s")
    wid = s * NC + c
    sl = pl.ds(s * rows_per_tile, rows_per_tile)

    pltpu.sync_copy(z2d, agg_sh.at[sl])
    if with_cnt:
      pltpu.sync_copy(z1d, cnt_sh.at[sl])
      ones16 = jnp.ones((LANES,), jnp.float32)

      def fill_ones(i, carry):
        ones_v[pl.ds(i * LANES, LANES)] = ones16
        return carry
      lax.fori_loop(0, CHUNK // LANES, fill_ones, 0)
    plsc.subcore_barrier()

    def sup_step(g, carry):
      pltpu.sync_copy(src3.at[wid, pl.ds(g * SUP, SUP)], src_v)
      pltpu.sync_copy(dst3.at[wid, pl.ds(g * SUP, SUP)], dst_v)

      def step(k, carry2):
        pltpu.async_copy(feat.at[src_v.at[k]], rows_v, sem).wait()
        pltpu.sync_copy(rows_v, agg_sh.at[dst_v.at[k]], add=True)
        if with_cnt:
          pltpu.sync_copy(ones_v, cnt_sh.at[dst_v.at[k]], add=True)
        return carry2
      return lax.fori_loop(0, SUP, step, carry)
    lax.fori_loop(0, nsup, sup_step, 0)

    plsc.subcore_barrier()
    pltpu.sync_copy(agg_sh.at[sl], agg_out.at[c, sl])
    if with_cnt:
      pltpu.sync_copy(
          cnt_sh.at[sl],
          cnt_out.at[pl.ds(c * n_pad + s * rows_per_tile, rows_per_tile)])

  return pl.kernel(body, out_type=tuple(out_type), mesh=mesh,
                   scratch_types=tuple(scratch))


def _tc_body(relu, p0, p1, cp, x, wl, wr, b, out):
  cnt = jnp.sum(cp[...], axis=0)
  inv = 1.0 / jnp.maximum(cnt, 1.0)
  mean = (p0[...] + p1[...]) * inv[:, None]
  acc = jnp.dot(mean, wl[...], preferred_element_type=jnp.float32)
  acc = acc + jnp.dot(x[...], wr[...], preferred_element_type=jnp.float32)
  acc = acc + b[...]
  if relu:
    acc = jnp.maximum(acc, 0.0)
  out[...] = acc


def _make_tc_layer(n, d, relu, blk=1024):
  grid = n // blk
  return pl.pallas_call(
      functools.partial(_tc_body, relu),
      grid=(grid,),
      in_specs=[
          pl.BlockSpec((blk, d), lambda i: (i, 0)),
          pl.BlockSpec((blk, d), lambda i: (i, 0)),
          pl.BlockSpec((NC, blk), lambda i: (0, i)),
          pl.BlockSpec((blk, d), lambda i: (i, 0)),
          pl.BlockSpec((d, d), lambda i: (0, 0)),
          pl.BlockSpec((d, d), lambda i: (0, 0)),
          pl.BlockSpec((1, d), lambda i: (0, 0)),
      ],
      out_specs=pl.BlockSpec((blk, d), lambda i: (i, 0)),
      out_shape=jax.ShapeDtypeStruct((n, d), jnp.float32),
  )


def kernel(x, edge_index, W1l, W1r, b1, W2l, W2r, b2):
  n, d = x.shape
  e = edge_index.shape[1]
  n_pad = ((n + 1 + 2047) // 2048) * 2048
  rows_per_tile = n_pad // NS
  steps = -(-e // (NW * CHUNK * SUP)) * SUP
  e_pad = steps * NW * CHUNK

  src = jnp.pad(edge_index[0], (0, e_pad - e)).reshape(NW, steps, CHUNK)
  dst = jnp.pad(edge_index[1], (0, e_pad - e),
                constant_values=n).reshape(NW, steps, CHUNK)
  z2d = jnp.zeros((rows_per_tile, d), jnp.float32)
  z1d = jnp.zeros((rows_per_tile,), jnp.float32)
  xp = jnp.pad(x, ((0, n_pad - n), (0, 0)))
  b1r = b1.reshape(1, d)
  b2r = b2.reshape(1, d)

  sc1 = _make_sc_agg(n_pad, d, steps, with_cnt=True)
  sc2 = _make_sc_agg(n_pad, d, steps, with_cnt=False)
  tc1 = _make_tc_layer(n_pad, d, relu=True)
  tc2 = _make_tc_layer(n_pad, d, relu=False)

  agg1, cnt = sc1(x, src, dst, z2d, z1d)
  cnt2 = cnt.reshape(NC, n_pad)
  h = tc1(agg1[0], agg1[1], cnt2, xp, W1l, W1r, b1r)
  (agg2,) = sc2(h, src, dst, z2d, z1d)
  out = tc2(agg2[0], agg2[1], cnt2, h, W2l, W2r, b2r)
  return out[:n]


# speedup vs baseline: 3.5218x; 1.0621x over previous
import functools

import jax
import jax.numpy as jnp
from jax import lax
from jax.experimental import pallas as pl
from jax.experimental.pallas import tpu as pltpu
from jax.experimental.pallas import tpu_sc as plsc

NC = 2
NS = 16
NW = NC * NS
LANES = 16
CHUNK = 128
SUP = 8


def _make_sc_agg(n_pad, d, steps, with_cnt):
  rows_per_tile = n_pad // NS
  nsup = steps // SUP
  mesh = plsc.VectorSubcoreMesh(
      core_axis_name="c", subcore_axis_name="s",
      num_cores=NC, num_subcores=NS)

  out_type = [jax.ShapeDtypeStruct((NC, n_pad, d), jnp.float32)]
  scratch = [
      pltpu.VMEM_SHARED((n_pad, d), jnp.float32),
      pltpu.VMEM((SUP, CHUNK), jnp.int32),
      pltpu.VMEM((SUP, CHUNK), jnp.int32),
      pltpu.VMEM((2, CHUNK, d), jnp.float32),
      pltpu.SemaphoreType.DMA,
      pltpu.SemaphoreType.DMA,
      pltpu.SemaphoreType.DMA,
  ]
  if with_cnt:
    out_type.append(jax.ShapeDtypeStruct((NC * n_pad,), jnp.float32))
    scratch.append(pltpu.VMEM_SHARED((n_pad,), jnp.float32))
    scratch.append(pltpu.VMEM((CHUNK,), jnp.float32))

  def body(feat, src3, dst3, z2d, z1d, *rest):
    if with_cnt:
      (agg_out, cnt_out, agg_sh, src_v, dst_v, rows_v, semg, sems, semc,
       cnt_sh, ones_v) = rest
    else:
      agg_out, agg_sh, src_v, dst_v, rows_v, semg, sems, semc = rest
    c = lax.axis_index("c")
    s = lax.axis_index("s")
    wid = s * NC + c
    sl = pl.ds(s * rows_per_tile, rows_per_tile)

    pltpu.sync_copy(z2d, agg_sh.at[sl])
    if with_cnt:
      pltpu.sync_copy(z1d, cnt_sh.at[sl])
      ones16 = jnp.ones((LANES,), jnp.float32)

      def fill_ones(i, carry):
        ones_v[pl.ds(i * LANES, LANES)] = ones16
        return carry
      lax.fori_loop(0, CHUNK // LANES, fill_ones, 0)
    plsc.subcore_barrier()

    def sup_step(g, carry):
      pltpu.sync_copy(src3.at[wid, pl.ds(g * SUP, SUP)], src_v)
      pltpu.sync_copy(dst3.at[wid, pl.ds(g * SUP, SUP)], dst_v)

      hg = [None] * SUP
      hs = [None] * SUP
      hc = [None] * SUP
      hg[0] = pltpu.async_copy(feat.at[src_v.at[0]], rows_v.at[0], semg)
      for k in range(SUP):
        hg[k].wait()
        if k > 0:
          hs[k - 1].wait()
        if k + 1 < SUP:
          hg[k + 1] = pltpu.async_copy(
              feat.at[src_v.at[k + 1]], rows_v.at[(k + 1) % 2], semg)
        hs[k] = pltpu.async_copy(
            rows_v.at[k % 2], agg_sh.at[dst_v.at[k]], sems, add=True)
        if with_cnt:
          hc[k] = pltpu.async_copy(
              ones_v, cnt_sh.at[dst_v.at[k]], semc, add=True)
      hs[SUP - 1].wait()
      if with_cnt:
        for k in range(SUP):
          hc[k].wait()
      return carry
    lax.fori_loop(0, nsup, sup_step, 0)

    plsc.subcore_barrier()
    pltpu.sync_copy(agg_sh.at[sl], agg_out.at[c, sl])
    if with_cnt:
      pltpu.sync_copy(
          cnt_sh.at[sl],
          cnt_out.at[pl.ds(c * n_pad + s * rows_per_tile, rows_per_tile)])

  return pl.kernel(body, out_type=tuple(out_type), mesh=mesh,
                   scratch_types=tuple(scratch))


def _tc_body(relu, p0, p1, cp, x, wl, wr, b, out):
  cnt = jnp.sum(cp[...], axis=0)
  inv = 1.0 / jnp.maximum(cnt, 1.0)
  mean = (p0[...] + p1[...]) * inv[:, None]
  acc = jnp.dot(mean, wl[...], preferred_element_type=jnp.float32)
  acc = acc + jnp.dot(x[...], wr[...], preferred_element_type=jnp.float32)
  acc = acc + b[...]
  if relu:
    acc = jnp.maximum(acc, 0.0)
  out[...] = acc


def _make_tc_layer(n, d, relu, blk=1024):
  grid = n // blk
  return pl.pallas_call(
      functools.partial(_tc_body, relu),
      grid=(grid,),
      in_specs=[
          pl.BlockSpec((blk, d), lambda i: (i, 0)),
          pl.BlockSpec((blk, d), lambda i: (i, 0)),
          pl.BlockSpec((NC, blk), lambda i: (0, i)),
          pl.BlockSpec((blk, d), lambda i: (i, 0)),
          pl.BlockSpec((d, d), lambda i: (0, 0)),
          pl.BlockSpec((d, d), lambda i: (0, 0)),
          pl.BlockSpec((1, d), lambda i: (0, 0)),
      ],
      out_specs=pl.BlockSpec((blk, d), lambda i: (i, 0)),
      out_shape=jax.ShapeDtypeStruct((n, d), jnp.float32),
  )


def kernel(x, edge_index, W1l, W1r, b1, W2l, W2r, b2):
  n, d = x.shape
  e = edge_index.shape[1]
  n_pad = ((n + 1 + 2047) // 2048) * 2048
  rows_per_tile = n_pad // NS
  steps = -(-e // (NW * CHUNK * SUP)) * SUP
  e_pad = steps * NW * CHUNK

  src = jnp.pad(edge_index[0], (0, e_pad - e)).reshape(NW, steps, CHUNK)
  dst = jnp.pad(edge_index[1], (0, e_pad - e),
                constant_values=n).reshape(NW, steps, CHUNK)
  z2d = jnp.zeros((rows_per_tile, d), jnp.float32)
  z1d = jnp.zeros((rows_per_tile,), jnp.float32)
  xp = jnp.pad(x, ((0, n_pad - n), (0, 0)))
  b1r = b1.reshape(1, d)
  b2r = b2.reshape(1, d)

  sc1 = _make_sc_agg(n_pad, d, steps, with_cnt=True)
  sc2 = _make_sc_agg(n_pad, d, steps, with_cnt=False)
  tc1 = _make_tc_layer(n_pad, d, relu=True)
  tc2 = _make_tc_layer(n_pad, d, relu=False)

  agg1, cnt = sc1(x, src, dst, z2d, z1d)
  cnt2 = cnt.reshape(NC, n_pad)
  h = tc1(agg1[0], agg1[1], cnt2, xp, W1l, W1r, b1r)
  (agg2,) = sc2(h, src, dst, z2d, z1d)
  out = tc2(agg2[0], agg2[1], cnt2, h, W2l, W2r, b2r)
  return out[:n]


# speedup vs baseline: 4.0584x; 1.1524x over previous
import functools

import jax
import jax.numpy as jnp
from jax import lax
from jax.experimental import pallas as pl
from jax.experimental.pallas import tpu as pltpu
from jax.experimental.pallas import tpu_sc as plsc

NC = 2
NS = 16
NW = NC * NS
LANES = 16
CHUNK = 128
SUP = 8


def _make_sc_agg(n_pad, d, steps0, steps1, with_cnt):
  rows_per_tile = n_pad // NS
  mesh = plsc.VectorSubcoreMesh(
      core_axis_name="c", subcore_axis_name="s",
      num_cores=NC, num_subcores=NS)

  out_type = [jax.ShapeDtypeStruct((NC, n_pad, d), jnp.float32)]
  scratch = [
      pltpu.VMEM_SHARED((n_pad, d), jnp.float32),
      pltpu.VMEM((SUP, CHUNK), jnp.int32),
      pltpu.VMEM((SUP, CHUNK), jnp.int32),
      pltpu.VMEM((2, CHUNK, d), jnp.float32),
      pltpu.SemaphoreType.DMA,
      pltpu.SemaphoreType.DMA,
      pltpu.SemaphoreType.DMA,
  ]
  if with_cnt:
    out_type.append(jax.ShapeDtypeStruct((NC * n_pad,), jnp.float32))
    scratch.append(pltpu.VMEM_SHARED((n_pad,), jnp.float32))
    scratch.append(pltpu.VMEM((CHUNK,), jnp.float32))

  def body(feat, src3, dst3, z2d, z1d, *rest):
    if with_cnt:
      (agg_out, cnt_out, agg_sh, src_v, dst_v, rows_v, semg, sems, semc,
       cnt_sh, ones_v) = rest
    else:
      agg_out, agg_sh, src_v, dst_v, rows_v, semg, sems, semc = rest
    c = lax.axis_index("c")
    s = lax.axis_index("s")
    wid = s * NC + c
    sl = pl.ds(s * rows_per_tile, rows_per_tile)

    pltpu.sync_copy(z2d, agg_sh.at[sl])
    if with_cnt:
      pltpu.sync_copy(z1d, cnt_sh.at[sl])
      ones16 = jnp.ones((LANES,), jnp.float32)

      def fill_ones(i, carry):
        ones_v[pl.ds(i * LANES, LANES)] = ones16
        return carry
      lax.fori_loop(0, CHUNK // LANES, fill_ones, 0)
    plsc.subcore_barrier()

    nsup_c = jnp.where(c == 0, steps0 // SUP, steps1 // SUP)

    def sup_step(g, carry):
      pltpu.sync_copy(src3.at[wid, pl.ds(g * SUP, SUP)], src_v)
      pltpu.sync_copy(dst3.at[wid, pl.ds(g * SUP, SUP)], dst_v)

      hg = [None] * SUP
      hs = [None] * SUP
      hc = [None] * SUP
      hg[0] = pltpu.async_copy(feat.at[src_v.at[0]], rows_v.at[0], semg)
      for k in range(SUP):
        hg[k].wait()
        if k > 0:
          hs[k - 1].wait()
        if k + 1 < SUP:
          hg[k + 1] = pltpu.async_copy(
              feat.at[src_v.at[k + 1]], rows_v.at[(k + 1) % 2], semg)
        hs[k] = pltpu.async_copy(
            rows_v.at[k % 2], agg_sh.at[dst_v.at[k]], sems, add=True)
        if with_cnt:
          hc[k] = pltpu.async_copy(
              ones_v, cnt_sh.at[dst_v.at[k]], semc, add=True)
      hs[SUP - 1].wait()
      if with_cnt:
        for k in range(SUP):
          hc[k].wait()
      return carry
    lax.fori_loop(0, nsup_c, sup_step, 0)

    plsc.subcore_barrier()
    pltpu.sync_copy(agg_sh.at[sl], agg_out.at[c, sl])
    if with_cnt:
      pltpu.sync_copy(
          cnt_sh.at[sl],
          cnt_out.at[pl.ds(c * n_pad + s * rows_per_tile, rows_per_tile)])

  return pl.kernel(body, out_type=tuple(out_type), mesh=mesh,
                   scratch_types=tuple(scratch))


def _tc_body(relu, p0, p1, cp, x, wl, wr, b, out):
  cnt = jnp.sum(cp[...], axis=0)
  inv = 1.0 / jnp.maximum(cnt, 1.0)
  mean = (p0[...] + p1[...]) * inv[:, None]
  acc = jnp.dot(mean, wl[...], preferred_element_type=jnp.float32)
  acc = acc + jnp.dot(x[...], wr[...], preferred_element_type=jnp.float32)
  acc = acc + b[...]
  if relu:
    acc = jnp.maximum(acc, 0.0)
  out[...] = acc


def _make_tc_layer(n, d, relu, blk=1024):
  grid = n // blk
  return pl.pallas_call(
      functools.partial(_tc_body, relu),
      grid=(grid,),
      in_specs=[
          pl.BlockSpec((blk, d), lambda i: (i, 0)),
          pl.BlockSpec((blk, d), lambda i: (i, 0)),
          pl.BlockSpec((NC, blk), lambda i: (0, i)),
          pl.BlockSpec((blk, d), lambda i: (i, 0)),
          pl.BlockSpec((d, d), lambda i: (0, 0)),
          pl.BlockSpec((d, d), lambda i: (0, 0)),
          pl.BlockSpec((1, d), lambda i: (0, 0)),
      ],
      out_specs=pl.BlockSpec((blk, d), lambda i: (i, 0)),
      out_shape=jax.ShapeDtypeStruct((n, d), jnp.float32),
  )


def kernel(x, edge_index, W1l, W1r, b1, W2l, W2r, b2):
  n, d = x.shape
  e = edge_index.shape[1]
  n_pad = ((n + 1 + 2047) // 2048) * 2048
  rows_per_tile = n_pad // NS
  steps = -(-e // (NW * CHUNK * SUP)) * SUP
  e_pad = steps * NW * CHUNK
  steps0 = (2 * steps * 3 // 4) // SUP * SUP
  steps1 = 2 * steps - steps0

  def pack(flat, fill):
    flat = jnp.pad(flat, (0, e_pad - e), constant_values=fill)
    a = flat[:NS * steps0 * CHUNK].reshape(NS, steps0, CHUNK)
    b = flat[NS * steps0 * CHUNK:].reshape(NS, steps1, CHUNK)
    b = jnp.pad(b, ((0, 0), (0, steps0 - steps1), (0, 0)))
    return jnp.stack([a, b], axis=1).reshape(NW, steps0, CHUNK)

  src = pack(edge_index[0], 0)
  dst = pack(edge_index[1], n)
  z2d = jnp.zeros((rows_per_tile, d), jnp.float32)
  z1d = jnp.zeros((rows_per_tile,), jnp.float32)
  xp = jnp.pad(x, ((0, n_pad - n), (0, 0)))
  b1r = b1.reshape(1, d)
  b2r = b2.reshape(1, d)

  sc1 = _make_sc_agg(n_pad, d, steps0, steps1, with_cnt=True)
  sc2 = _make_sc_agg(n_pad, d, steps0, steps1, with_cnt=False)
  tc1 = _make_tc_layer(n_pad, d, relu=True)
  tc2 = _make_tc_layer(n_pad, d, relu=False)

  agg1, cnt = sc1(x, src, dst, z2d, z1d)
  cnt2 = cnt.reshape(NC, n_pad)
  h = tc1(agg1[0], agg1[1], cnt2, xp, W1l, W1r, b1r)
  (agg2,) = sc2(h, src, dst, z2d, z1d)
  out = tc2(agg2[0], agg2[1], cnt2, h, W2l, W2r, b2r)
  return out[:n]
